# baseline (speedup 1.0000x reference)
import functools

import jax
import jax.numpy as jnp
import jax.scipy.linalg as jsl
from jax.experimental import pallas as pl


def _tokenizer_kernel(x_ref, y_ref, w_ref, t_ref, o_ref, *, bb):
    classes = jax.lax.broadcasted_iota(jnp.int32, (1, 128), 1)
    for jb in range(bb):
        xj = x_ref[jb]
        acc = jax.lax.dot_general(
            xj, w_ref[...],
            dimension_numbers=(((1,), (0,)), ((), ())),
            preferred_element_type=jnp.float32,
        )
        for j in range(4):
            yjs = y_ref[jb, j][:, None]
            onehot = (yjs == classes).astype(jnp.float32)
            acc += jax.lax.dot_general(
                onehot, t_ref[j],
                dimension_numbers=(((1,), (0,)), ((), ())),
                preferred_element_type=jnp.float32,
            )
        o_ref[jb] = acc


@jax.jit
def kernel(x, y, W, b, table):
    B, N, D = x.shape
    H, _ = W.shape
    C = table.shape[0]
    BB = 8
    NCHUNK = 4
    BC = B // NCHUNK

    table_b = table + b[None, :]
    Wt = W.T
    Wd = jsl.block_diag(Wt, Wt, Wt, Wt)
    T4 = jnp.zeros((4, 128, 4 * H), jnp.float32)
    for j in range(4):
        T4 = T4.at[j, :C, j * H:(j + 1) * H].set(table_b)

    call = pl.pallas_call(
        functools.partial(_tokenizer_kernel, bb=BB),
        grid=(BC // BB,),
        in_specs=[
            pl.BlockSpec((BB, N // 4, 4 * D), lambda i: (i, 0, 0)),
            pl.BlockSpec((BB, 4, N // 4), lambda i: (i, 0, 0)),
            pl.BlockSpec((4 * D, 4 * H), lambda i: (0, 0)),
            pl.BlockSpec((4, 128, 4 * H), lambda i: (0, 0, 0)),
        ],
        out_specs=pl.BlockSpec((BB, N // 4, 4 * H), lambda i: (i, 0, 0)),
        out_shape=jax.ShapeDtypeStruct((BC, N // 4, 4 * H), jnp.float32),
    )

    chunks = []
    for c in range(NCHUNK):
        xc = jax.lax.slice_in_dim(x, c * BC, (c + 1) * BC, axis=0)
        yc = jax.lax.slice_in_dim(y, c * BC, (c + 1) * BC, axis=0)
        xpc = xc.reshape(BC, N // 4, 4 * D)
        ysc = jnp.transpose(yc.reshape(BC, N // 4, 4), (0, 2, 1))
        outc = call(xpc, ysc, Wd, T4)
        chunks.append(outc.reshape(BC, N, H))
    return jnp.concatenate(chunks, axis=0)


# speedup vs baseline: 2.0087x; 2.0087x over previous
import functools

import jax
import jax.numpy as jnp
import jax.scipy.linalg as jsl
from jax.experimental import pallas as pl


def _tokenizer_kernel(x_ref, y_ref, w_ref, t_ref, o_ref, *, bb):
    classes = jax.lax.broadcasted_iota(jnp.int32, (1, 128), 1)
    for jb in range(bb):
        xj = x_ref[jb]
        acc = jax.lax.dot_general(
            xj, w_ref[...],
            dimension_numbers=(((1,), (0,)), ((), ())),
            preferred_element_type=jnp.float32,
        )
        for j in range(4):
            yjs = y_ref[jb, j][:, None]
            onehot = (yjs == classes).astype(jnp.float32)
            acc += jax.lax.dot_general(
                onehot, t_ref[j],
                dimension_numbers=(((1,), (0,)), ((), ())),
                preferred_element_type=jnp.float32,
            )
        o_ref[jb] = acc.reshape(1024, 128)


@jax.jit
def kernel(x, y, W, b, table):
    B, N, D = x.shape
    H, _ = W.shape
    C = table.shape[0]
    BB = 8

    table_b = table + b[None, :]
    Wt = W.T
    Wd = jsl.block_diag(Wt, Wt, Wt, Wt)
    T4 = jnp.zeros((4, 128, 4 * H), jnp.float32)
    for j in range(4):
        T4 = T4.at[j, :C, j * H:(j + 1) * H].set(table_b)

    call = pl.pallas_call(
        functools.partial(_tokenizer_kernel, bb=BB),
        grid=(B // BB,),
        in_specs=[
            pl.BlockSpec((BB, N // 4, 4 * D), lambda i: (i, 0, 0)),
            pl.BlockSpec((BB, 4, N // 4), lambda i: (i, 0, 0)),
            pl.BlockSpec((4 * D, 4 * H), lambda i: (0, 0)),
            pl.BlockSpec((4, 128, 4 * H), lambda i: (0, 0, 0)),
        ],
        out_specs=pl.BlockSpec((BB, N // 2, 2 * H), lambda i: (i, 0, 0)),
        out_shape=jax.ShapeDtypeStruct((B, N // 2, 2 * H), jnp.float32),
    )

    xp = x.reshape(B, N // 4, 4 * D)
    ys = jnp.transpose(y.reshape(B, N // 4, 4), (0, 2, 1))
    out = call(xp, ys, Wd, T4)
    return out.reshape(B, N, H)
